# baseline (device time: 13917 ns/iter reference)
import jax
import jax.numpy as jnp
from jax import lax
from jax.experimental import pallas as pl
from jax.experimental.pallas import tpu as pltpu

N_DEV = 4


def kernel(x, w_mat):
    m_global, k_per = x.shape
    k_global, n = w_mat.shape
    m_per = m_global // N_DEV
    n_half = n // 2

    def body(x_hbm, w_hbm, out_hbm,
             xf32, xsend, xg, wvm, wbf, outv,
             xdma_sems, wdma_sem, odma_sems, send_sems, recv_sems):
        my = lax.axis_index("i")

        xorder = (2, 1, 3, 0)
        xdmas = []
        for i, off in enumerate(xorder):
            p = (my + off) % N_DEV
            c = pltpu.make_async_copy(
                x_hbm.at[pl.ds(p * m_per, m_per), :],
                xf32.at[pl.ds(p * m_per, m_per), :],
                xdma_sems.at[i],
            )
            c.start()
            xdmas.append(c)
        wdma = pltpu.make_async_copy(w_hbm, wvm, wdma_sem)
        wdma.start()

        barrier_sem = pltpu.get_barrier_semaphore()
        for off in range(1, N_DEV):
            peer = (my + off) % N_DEV
            pl.semaphore_signal(
                barrier_sem, inc=1, device_id=(peer,),
                device_id_type=pl.DeviceIdType.MESH,
            )
        pl.semaphore_wait(barrier_sem, N_DEV - 1)

        sends = []
        for i, off in enumerate(xorder[:3]):
            p = (my + off) % N_DEV
            xdmas[i].wait()
            xsend[i] = xf32[pl.ds(p * m_per, m_per), :].astype(jnp.bfloat16)
            rdma = pltpu.make_async_remote_copy(
                src_ref=xsend.at[i],
                dst_ref=xg.at[:, pl.ds(my * k_per, k_per)],
                send_sem=send_sems.at[i],
                recv_sem=recv_sems.at[N_DEV - off - 1],
                device_id=(p,),
                device_id_type=pl.DeviceIdType.MESH,
            )
            rdma.start()
            sends.append(rdma)

        xdmas[3].wait()
        xg[:, pl.ds(my * k_per, k_per)] = (
            xf32[pl.ds(my * m_per, m_per), :].astype(jnp.bfloat16)
        )

        wdma.wait()
        wbf[:, :] = wvm[:, :].astype(jnp.bfloat16)

        for r in (1, 3, 2):
            recv = pltpu.make_async_remote_copy(
                src_ref=xsend.at[0],
                dst_ref=xg.at[:, pl.ds(0, k_per)],
                send_sem=send_sems.at[0],
                recv_sem=recv_sems.at[r - 1],
                device_id=(my,),
                device_id_type=pl.DeviceIdType.MESH,
            )
            recv.wait_recv()

        outv[:, pl.ds(0, n_half)] = jnp.dot(
            xg[:, :], wbf[:, pl.ds(0, n_half)],
            preferred_element_type=jnp.float32,
        )
        o0 = pltpu.make_async_copy(
            outv.at[:, pl.ds(0, n_half)],
            out_hbm.at[:, pl.ds(0, n_half)],
            odma_sems.at[0],
        )
        o0.start()
        outv[:, pl.ds(n_half, n_half)] = jnp.dot(
            xg[:, :], wbf[:, pl.ds(n_half, n_half)],
            preferred_element_type=jnp.float32,
        )
        o1 = pltpu.make_async_copy(
            outv.at[:, pl.ds(n_half, n_half)],
            out_hbm.at[:, pl.ds(n_half, n_half)],
            odma_sems.at[1],
        )
        o1.start()
        o0.wait()
        o1.wait()

        for rdma in sends:
            rdma.wait_send()

    return pl.pallas_call(
        body,
        out_shape=jax.ShapeDtypeStruct((m_per, n), jnp.float32),
        in_specs=[
            pl.BlockSpec(memory_space=pl.ANY),
            pl.BlockSpec(memory_space=pl.ANY),
        ],
        out_specs=pl.BlockSpec(memory_space=pl.ANY),
        scratch_shapes=[
            pltpu.VMEM((m_global, k_per), jnp.float32),
            pltpu.VMEM((N_DEV - 1, m_per, k_per), jnp.bfloat16),
            pltpu.VMEM((m_per, k_global), jnp.bfloat16),
            pltpu.VMEM((k_global, n), jnp.float32),
            pltpu.VMEM((k_global, n), jnp.bfloat16),
            pltpu.VMEM((m_per, n), jnp.float32),
            pltpu.SemaphoreType.DMA((N_DEV,)),
            pltpu.SemaphoreType.DMA,
            pltpu.SemaphoreType.DMA((2,)),
            pltpu.SemaphoreType.DMA((N_DEV - 1,)),
            pltpu.SemaphoreType.DMA((N_DEV - 1,)),
        ],
        compiler_params=pltpu.CompilerParams(collective_id=0),
    )(x, w_mat)


# device time: 13866 ns/iter; 1.0037x vs baseline; 1.0037x over previous
import jax
import jax.numpy as jnp
from jax import lax
from jax.experimental import pallas as pl
from jax.experimental.pallas import tpu as pltpu

N_DEV = 4


def kernel(x, w_mat):
    m_global, k_per = x.shape
    k_global, n = w_mat.shape
    m_per = m_global // N_DEV
    n_half = n // 2

    def body(x_hbm, w_hbm, out_hbm,
             xf32, xsend, xg, wvm, wbf, outv,
             xdma_sems, wdma_sem, odma_sems, send_sems, recv_sems):
        my = lax.axis_index("i")

        xorder = (2, 1, 3, 0)
        xdmas = []
        for i, off in enumerate(xorder):
            p = (my + off) % N_DEV
            c = pltpu.make_async_copy(
                x_hbm.at[pl.ds(p * m_per, m_per), :],
                xf32.at[pl.ds(p * m_per, m_per), :],
                xdma_sems.at[i],
            )
            c.start()
            xdmas.append(c)
        wdma = pltpu.make_async_copy(w_hbm, wvm, wdma_sem)
        wdma.start()

        barrier_sem = pltpu.get_barrier_semaphore()
        for off in range(1, N_DEV):
            peer = (my + off) % N_DEV
            pl.semaphore_signal(
                barrier_sem, inc=1, device_id=(peer,),
                device_id_type=pl.DeviceIdType.MESH,
            )
        pl.semaphore_wait(barrier_sem, N_DEV - 1)

        sends = []
        for i, off in enumerate(xorder[:3]):
            p = (my + off) % N_DEV
            xdmas[i].wait()
            xsend[i] = xf32[pl.ds(p * m_per, m_per), :].astype(jnp.bfloat16)
            rdma = pltpu.make_async_remote_copy(
                src_ref=xsend.at[i],
                dst_ref=xg.at[:, pl.ds(my * k_per, k_per)],
                send_sem=send_sems.at[i],
                recv_sem=recv_sems.at[N_DEV - off - 1],
                device_id=(p,),
                device_id_type=pl.DeviceIdType.MESH,
            )
            rdma.start()
            sends.append(rdma)

        xdmas[3].wait()
        xg[:, pl.ds(my * k_per, k_per)] = (
            xf32[pl.ds(my * m_per, m_per), :].astype(jnp.bfloat16)
        )

        wdma.wait()
        wbf[:, :] = wvm[:, :].astype(jnp.bfloat16)

        for r in (1, 3, 2):
            recv = pltpu.make_async_remote_copy(
                src_ref=xsend.at[0],
                dst_ref=xg.at[:, pl.ds(0, k_per)],
                send_sem=send_sems.at[0],
                recv_sem=recv_sems.at[r - 1],
                device_id=(my,),
                device_id_type=pl.DeviceIdType.MESH,
            )
            recv.wait_recv()

        outv[:, pl.ds(0, n_half)] = jnp.dot(
            xg[:, :], wbf[:, pl.ds(0, n_half)],
            preferred_element_type=jnp.float32,
        )
        o0 = pltpu.make_async_copy(
            outv.at[:, pl.ds(0, n_half)],
            out_hbm.at[:, pl.ds(0, n_half)],
            odma_sems.at[0],
        )
        o0.start()
        outv[:, pl.ds(n_half, n_half)] = jnp.dot(
            xg[:, :], wbf[:, pl.ds(n_half, n_half)],
            preferred_element_type=jnp.float32,
        )
        o1 = pltpu.make_async_copy(
            outv.at[:, pl.ds(n_half, n_half)],
            out_hbm.at[:, pl.ds(n_half, n_half)],
            odma_sems.at[1],
        )
        o1.start()
        o0.wait()
        o1.wait()

        for rdma in sends:
            rdma.wait_send()

    return pl.pallas_call(
        body,
        out_shape=jax.ShapeDtypeStruct((m_per, n), jnp.float32),
        in_specs=[
            pl.BlockSpec(memory_space=pltpu.MemorySpace.HBM),
            pl.BlockSpec(memory_space=pltpu.MemorySpace.HBM),
        ],
        out_specs=pl.BlockSpec(memory_space=pltpu.MemorySpace.HBM),
        scratch_shapes=[
            pltpu.VMEM((m_global, k_per), jnp.float32),
            pltpu.VMEM((N_DEV - 1, m_per, k_per), jnp.bfloat16),
            pltpu.VMEM((m_per, k_global), jnp.bfloat16),
            pltpu.VMEM((k_global, n), jnp.float32),
            pltpu.VMEM((k_global, n), jnp.bfloat16),
            pltpu.VMEM((m_per, n), jnp.float32),
            pltpu.SemaphoreType.DMA((N_DEV,)),
            pltpu.SemaphoreType.DMA,
            pltpu.SemaphoreType.DMA((2,)),
            pltpu.SemaphoreType.DMA((N_DEV - 1,)),
            pltpu.SemaphoreType.DMA((N_DEV - 1,)),
        ],
        compiler_params=pltpu.CompilerParams(collective_id=0),
    )(x, w_mat)


# device time: 13756 ns/iter; 1.0117x vs baseline; 1.0080x over previous
import jax
import jax.numpy as jnp
from jax import lax
from jax.experimental import pallas as pl
from jax.experimental.pallas import tpu as pltpu

N_DEV = 4


def kernel(x, w_mat):
    m_global, k_per = x.shape
    k_global, n = w_mat.shape
    m_per = m_global // N_DEV
    n_half = n // 2

    def body(x_hbm, w_hbm, out_hbm,
             xf32, xsend, xg, wvm, wbf, outv, vmem_filler,
             xdma_sems, wdma_sem, odma_sems, send_sems, recv_sems):
        del vmem_filler
        my = lax.axis_index("i")

        xorder = (2, 1, 3, 0)
        xdmas = []
        for i, off in enumerate(xorder):
            p = (my + off) % N_DEV
            c = pltpu.make_async_copy(
                x_hbm.at[pl.ds(p * m_per, m_per), :],
                xf32.at[pl.ds(p * m_per, m_per), :],
                xdma_sems.at[i],
            )
            c.start()
            xdmas.append(c)
        wdma = pltpu.make_async_copy(w_hbm, wvm, wdma_sem)
        wdma.start()

        barrier_sem = pltpu.get_barrier_semaphore()
        for off in range(1, N_DEV):
            peer = (my + off) % N_DEV
            pl.semaphore_signal(
                barrier_sem, inc=1, device_id=(peer,),
                device_id_type=pl.DeviceIdType.MESH,
            )
        pl.semaphore_wait(barrier_sem, N_DEV - 1)

        sends = []
        for i, off in enumerate(xorder[:3]):
            p = (my + off) % N_DEV
            xdmas[i].wait()
            xsend[i] = xf32[pl.ds(p * m_per, m_per), :].astype(jnp.bfloat16)
            rdma = pltpu.make_async_remote_copy(
                src_ref=xsend.at[i],
                dst_ref=xg.at[:, pl.ds(my * k_per, k_per)],
                send_sem=send_sems.at[i],
                recv_sem=recv_sems.at[N_DEV - off - 1],
                device_id=(p,),
                device_id_type=pl.DeviceIdType.MESH,
            )
            rdma.start()
            sends.append(rdma)

        xdmas[3].wait()
        xg[:, pl.ds(my * k_per, k_per)] = (
            xf32[pl.ds(my * m_per, m_per), :].astype(jnp.bfloat16)
        )

        wdma.wait()
        wbf[:, :] = wvm[:, :].astype(jnp.bfloat16)

        for r in (1, 3, 2):
            recv = pltpu.make_async_remote_copy(
                src_ref=xsend.at[0],
                dst_ref=xg.at[:, pl.ds(0, k_per)],
                send_sem=send_sems.at[0],
                recv_sem=recv_sems.at[r - 1],
                device_id=(my,),
                device_id_type=pl.DeviceIdType.MESH,
            )
            recv.wait_recv()

        outv[:, pl.ds(0, n_half)] = jnp.dot(
            xg[:, :], wbf[:, pl.ds(0, n_half)],
            preferred_element_type=jnp.float32,
        )
        o0 = pltpu.make_async_copy(
            outv.at[:, pl.ds(0, n_half)],
            out_hbm.at[:, pl.ds(0, n_half)],
            odma_sems.at[0],
        )
        o0.start()
        outv[:, pl.ds(n_half, n_half)] = jnp.dot(
            xg[:, :], wbf[:, pl.ds(n_half, n_half)],
            preferred_element_type=jnp.float32,
        )
        o1 = pltpu.make_async_copy(
            outv.at[:, pl.ds(n_half, n_half)],
            out_hbm.at[:, pl.ds(n_half, n_half)],
            odma_sems.at[1],
        )
        o1.start()
        o0.wait()
        o1.wait()

        for rdma in sends:
            rdma.wait_send()

    return pl.pallas_call(
        body,
        out_shape=jax.ShapeDtypeStruct((m_per, n), jnp.float32),
        in_specs=[
            pl.BlockSpec(memory_space=pltpu.MemorySpace.HBM),
            pl.BlockSpec(memory_space=pltpu.MemorySpace.HBM),
        ],
        out_specs=pl.BlockSpec(memory_space=pltpu.MemorySpace.HBM),
        scratch_shapes=[
            pltpu.VMEM((m_global, k_per), jnp.float32),
            pltpu.VMEM((N_DEV - 1, m_per, k_per), jnp.bfloat16),
            pltpu.VMEM((m_per, k_global), jnp.bfloat16),
            pltpu.VMEM((k_global, n), jnp.float32),
            pltpu.VMEM((k_global, n), jnp.bfloat16),
            pltpu.VMEM((m_per, n), jnp.float32),
            pltpu.VMEM((13, 1024, 1024), jnp.float32),
            pltpu.SemaphoreType.DMA((N_DEV,)),
            pltpu.SemaphoreType.DMA,
            pltpu.SemaphoreType.DMA((2,)),
            pltpu.SemaphoreType.DMA((N_DEV - 1,)),
            pltpu.SemaphoreType.DMA((N_DEV - 1,)),
        ],
        compiler_params=pltpu.CompilerParams(collective_id=0),
    )(x, w_mat)


# device time: 13399 ns/iter; 1.0387x vs baseline; 1.0266x over previous
import jax
import jax.numpy as jnp
from jax import lax
from jax.experimental import pallas as pl
from jax.experimental.pallas import tpu as pltpu

N_DEV = 4


def kernel(x, w_mat):
    m_global, k_per = x.shape
    k_global, n = w_mat.shape
    m_per = m_global // N_DEV
    n_half = n // 2

    xb = x.astype(jnp.bfloat16)
    wb = w_mat.astype(jnp.bfloat16)

    def body(xb_ref, wb_ref, out_ref, xg, send_sems, recv_sems):
        my = lax.axis_index("i")

        barrier_sem = pltpu.get_barrier_semaphore()
        for off in range(1, N_DEV):
            peer = (my + off) % N_DEV
            pl.semaphore_signal(
                barrier_sem, inc=1, device_id=(peer,),
                device_id_type=pl.DeviceIdType.MESH,
            )
        pl.semaphore_wait(barrier_sem, N_DEV - 1)

        sends = []
        for i, off in enumerate((2, 1, 3)):
            peer = (my + off) % N_DEV
            rdma = pltpu.make_async_remote_copy(
                src_ref=xb_ref.at[pl.ds(peer * m_per, m_per), :],
                dst_ref=xg.at[:, pl.ds(my * k_per, k_per)],
                send_sem=send_sems.at[i],
                recv_sem=recv_sems.at[N_DEV - off - 1],
                device_id=(peer,),
                device_id_type=pl.DeviceIdType.MESH,
            )
            rdma.start()
            sends.append(rdma)

        xg[:, pl.ds(my * k_per, k_per)] = xb_ref[pl.ds(my * m_per, m_per), :]

        for r in (1, 3, 2):
            recv = pltpu.make_async_remote_copy(
                src_ref=xb_ref.at[pl.ds(0, m_per), :],
                dst_ref=xg.at[:, pl.ds(0, k_per)],
                send_sem=send_sems.at[0],
                recv_sem=recv_sems.at[r - 1],
                device_id=(my,),
                device_id_type=pl.DeviceIdType.MESH,
            )
            recv.wait_recv()

        out_ref[:, pl.ds(0, n_half)] = jnp.dot(
            xg[:, :], wb_ref[:, pl.ds(0, n_half)],
            preferred_element_type=jnp.float32,
        )
        out_ref[:, pl.ds(n_half, n_half)] = jnp.dot(
            xg[:, :], wb_ref[:, pl.ds(n_half, n_half)],
            preferred_element_type=jnp.float32,
        )

        for rdma in sends:
            rdma.wait_send()

    return pl.pallas_call(
        body,
        out_shape=jax.ShapeDtypeStruct((m_per, n), jnp.float32),
        in_specs=[
            pl.BlockSpec(memory_space=pltpu.MemorySpace.VMEM),
            pl.BlockSpec(memory_space=pltpu.MemorySpace.VMEM),
        ],
        out_specs=pl.BlockSpec(memory_space=pltpu.MemorySpace.VMEM),
        scratch_shapes=[
            pltpu.VMEM((m_per, k_global), jnp.bfloat16),
            pltpu.SemaphoreType.DMA((N_DEV - 1,)),
            pltpu.SemaphoreType.DMA((N_DEV - 1,)),
        ],
        compiler_params=pltpu.CompilerParams(collective_id=0),
    )(xb, wb)
